# Initial kernel scaffold; baseline (speedup 1.0000x reference)
import jax
import jax.numpy as jnp
from jax.experimental import pallas as pl


def kernel(x_in, edge_index, We, be, Wd, bd, W1, b1, W2, b2, W3, b3):
    raise NotImplementedError("write your pallas kernel here")


# speedup vs baseline: 6.1605x; 6.1605x over previous
import functools

import jax
import jax.numpy as jnp
from jax import lax
from jax.experimental import pallas as pl
from jax.experimental.pallas import tpu as pltpu
from jax.experimental.pallas import tpu_sc as plsc

NC = 2
NS = 16
NW = NC * NS
CH = 128


def _elu01(t):
    return jnp.where(t > 0, t, 0.1 * (jnp.exp(t) - 1.0))


def _make_sc_agg(n, n_acc, epad, feat, with_deg):
    cpw = epad // (NW * CH)
    rpt = (n // NS) // 8 * 8
    rtail = n - NS * rpt
    zch = n_acc // (NS * CH)

    out_type = [jax.ShapeDtypeStruct((NC, n, feat), jnp.float32)]
    scratch = [
        pltpu.VMEM_SHARED((n_acc, feat), jnp.float32),
        pltpu.VMEM((CH,), jnp.int32),
        pltpu.VMEM((CH,), jnp.int32),
        pltpu.VMEM((CH, feat), jnp.float32),
        pltpu.SemaphoreType.DMA,
    ]
    if with_deg:
        out_type.append(jax.ShapeDtypeStruct((NC, n, 16), jnp.float32))
        scratch += [
            pltpu.VMEM_SHARED((n_acc, 16), jnp.float32),
            pltpu.VMEM((CH, 16), jnp.float32),
        ]

    def body(x_hbm, src_hbm, dst_hbm, *rest):
        if with_deg:
            out_hbm, deg_hbm, acc, srcb, dstb, rowsb, sem, dacc, onesb = rest
        else:
            out_hbm, acc, srcb, dstb, rowsb, sem = rest
        c = lax.axis_index("c")
        s = lax.axis_index("s")
        wid = s * NC + c

        z16 = jnp.zeros((16,), jnp.float32)

        @pl.loop(0, CH)
        def _(r):
            for j in range(feat // 16):
                rowsb[r, pl.ds(j * 16, 16)] = z16

        zbase = s * (zch * CH)

        @pl.loop(0, zch)
        def _(z):
            pltpu.sync_copy(rowsb, acc.at[pl.ds(zbase + z * CH, CH)])

        if with_deg:
            @pl.loop(0, CH)
            def _(r):
                onesb[r, pl.ds(0, 16)] = z16

            @pl.loop(0, zch)
            def _(z):
                pltpu.sync_copy(onesb, dacc.at[pl.ds(zbase + z * CH, CH)])

            one16 = jnp.ones((16,), jnp.float32)

            @pl.loop(0, CH)
            def _(r):
                onesb[r, pl.ds(0, 16)] = one16

        plsc.subcore_barrier()

        base = wid * cpw

        @pl.loop(0, cpw)
        def _(k):
            off = (base + k) * CH
            pltpu.sync_copy(src_hbm.at[pl.ds(off, CH)], srcb)
            pltpu.sync_copy(dst_hbm.at[pl.ds(off, CH)], dstb)
            pltpu.async_copy(x_hbm.at[srcb], rowsb, sem).wait()
            pltpu.sync_copy(rowsb, acc.at[dstb], add=True)
            if with_deg:
                pltpu.sync_copy(onesb, dacc.at[dstb], add=True)

        plsc.subcore_barrier()

        rb = s * rpt
        pltpu.sync_copy(acc.at[pl.ds(rb, rpt)], out_hbm.at[c, pl.ds(rb, rpt)])
        if with_deg:
            pltpu.sync_copy(dacc.at[pl.ds(rb, rpt)],
                            deg_hbm.at[c, pl.ds(rb, rpt)])
        if rtail:
            @pl.when(s == NS - 1)
            def _():
                tb = NS * rpt
                pltpu.sync_copy(acc.at[pl.ds(tb, rtail)],
                                out_hbm.at[c, pl.ds(tb, rtail)])
                if with_deg:
                    pltpu.sync_copy(dacc.at[pl.ds(tb, rtail)],
                                    deg_hbm.at[c, pl.ds(tb, rtail)])

    return pl.kernel(
        body,
        out_type=out_type,
        mesh=plsc.VectorSubcoreMesh(core_axis_name="c", subcore_axis_name="s"),
        scratch_types=scratch,
        compiler_params=pltpu.CompilerParams(use_tc_tiling_on_sc=False),
    )


R = 1000


def _tc_recover_body(x, We, be, Wd, bd, out):
    xc = jnp.dot(x[...], We[...], preferred_element_type=jnp.float32)
    xc = xc + be[...][None, :]
    out[...] = jnp.dot(xc, Wd[...], preferred_element_type=jnp.float32) \
        + bd[...][None, :]


def _tc_mid_body(p0, p1, d0, d1, We, W1, W2, b1, b2, be, W3a, W3b, out):
    agg = p0[...] + p1[...]
    deg = (d0[...] + d1[...])[:, 0:1]
    m1 = jnp.dot(We[...], W1[...], preferred_element_type=jnp.float32)
    v1 = jnp.dot(be[...][None, :], W1[...], preferred_element_type=jnp.float32)
    h2 = jnp.dot(agg, W2[...], preferred_element_type=jnp.float32) \
        + b2[...][None, :]
    h1 = jnp.dot(agg, m1, preferred_element_type=jnp.float32) \
        + deg * v1 + b1[...][None, :]
    out[...] = jnp.dot(_elu01(h2), W3a[...], preferred_element_type=jnp.float32) \
        + jnp.dot(_elu01(h1), W3b[...], preferred_element_type=jnp.float32)


def _tc_softmax_body(q0, q1, b3, out):
    x3 = q0[...] + q1[...] + b3[...][None, :]
    m = jnp.max(x3, axis=1, keepdims=True)
    e = x3 - m
    lse = jnp.log(jnp.sum(jnp.exp(e), axis=1, keepdims=True))
    out[...] = e - lse


def _row_spec(f):
    return pl.BlockSpec((R, f), lambda i: (i, 0))


def _full_spec(shape):
    nd = len(shape)
    return pl.BlockSpec(shape, (lambda i: (0,) * nd))


def kernel(x_in, edge_index, We, be, Wd, bd, W1, b1, W2, b2, W3, b3):
    n, nfeat = x_in.shape
    e = edge_index.shape[1]
    assert n % NS == 0
    src = edge_index[0]
    dst = edge_index[1]
    epad = -(-e // (NW * CH)) * (NW * CH)
    if epad != e:
        pad = epad - e
        src = jnp.concatenate([src, jnp.zeros((pad,), jnp.int32)])
        dst = jnp.concatenate([dst, jnp.full((pad,), n, jnp.int32)])
    n_acc = -(-(n + 1) // (NS * CH)) * (NS * CH)

    P, D = _make_sc_agg(n, n_acc, epad, nfeat, True)(x_in, src, dst)

    grid = n // R

    x_recover = pl.pallas_call(
        _tc_recover_body,
        grid=(grid,),
        in_specs=[_row_spec(nfeat), _full_spec(We.shape), _full_spec(be.shape),
                  _full_spec(Wd.shape), _full_spec(bd.shape)],
        out_specs=_row_spec(nfeat),
        out_shape=jax.ShapeDtypeStruct((n, nfeat), jnp.float32),
    )(x_in, We, be, Wd, bd)

    support3 = pl.pallas_call(
        _tc_mid_body,
        grid=(grid,),
        in_specs=[_row_spec(nfeat), _row_spec(nfeat),
                  _row_spec(16), _row_spec(16),
                  _full_spec(We.shape), _full_spec(W1.shape),
                  _full_spec(W2.shape), _full_spec(b1.shape),
                  _full_spec(b2.shape), _full_spec(be.shape),
                  _full_spec((nfeat, W3.shape[1])),
                  _full_spec((nfeat, W3.shape[1]))],
        out_specs=_row_spec(W3.shape[1]),
        out_shape=jax.ShapeDtypeStruct((n, W3.shape[1]), jnp.float32),
    )(P[0], P[1], D[0], D[1], We, W1, W2, b1, b2, be, W3[:nfeat], W3[nfeat:])

    Q, = _make_sc_agg(n, n_acc, epad, W3.shape[1], False)(support3, src, dst)

    x_out = pl.pallas_call(
        _tc_softmax_body,
        grid=(grid,),
        in_specs=[_row_spec(W3.shape[1]), _row_spec(W3.shape[1]),
                  _full_spec(b3.shape)],
        out_specs=_row_spec(W3.shape[1]),
        out_shape=jax.ShapeDtypeStruct((n, W3.shape[1]), jnp.float32),
    )(Q[0], Q[1], b3)

    return (x_out, x_recover)
